# baseline (speedup 1.0000x reference)
import jax
import jax.numpy as jnp
from jax import lax
from jax.experimental import pallas as pl
from jax.experimental.pallas import tpu as pltpu
from jax.experimental.pallas import tpu_sc as plsc

_BATCH = 16384
_FIELDS = 26
_DIM = 32
_VOCAB = 1000000
_TOTAL = _BATCH * _FIELDS
_NUM_CORES = 2
_NUM_SUBCORES = 16
_NW = _NUM_CORES * _NUM_SUBCORES
_B_PER_W = _TOTAL // _NW
_N_CHUNKS = 8
_CHUNK = _B_PER_W // _N_CHUNKS
_NBUF = 2

_W = 800
_NWIN = _VOCAB // _W
_WIN_FULL = _NWIN // _NW
_WIN_EXTRA = _NWIN % _NW
_WGRP = _W // 16


def _transpose_body(tt_hbm, trm_hbm, slab_v, tbuf_v, sem):
    wid = lax.axis_index("s") * _NUM_CORES + lax.axis_index("c")
    n_win = jnp.where(wid < _WIN_EXTRA, _WIN_FULL + 1, _WIN_FULL)
    lanes = lax.iota(jnp.int32, 16)

    def window(k, carry):
        w0 = (wid + _NW * k) * _W
        pltpu.sync_copy(tt_hbm.at[:, pl.ds(w0, _W)], slab_v)

        def group(g, c2):
            row = g * 16 + lanes
            for d in range(_DIM):
                vec = slab_v[d, pl.ds(g * 16, 16)]
                plsc.store_scatter(tbuf_v, [row * _DIM + d], vec)
            return c2

        lax.fori_loop(0, _WGRP, group, 0)
        pltpu.sync_copy(tbuf_v, trm_hbm.at[pl.ds(w0 * _DIM, _W * _DIM)])
        return carry

    lax.fori_loop(0, n_win, window, 0)


def _gather_body(idx_hbm, trm_hbm, out_hbm, idx_v, rows0, rows1, s0, s1):
    wid = lax.axis_index("s") * _NUM_CORES + lax.axis_index("c")
    base = wid * _B_PER_W
    rows = (rows0, rows1)
    sems = (s0, s1)

    pltpu.sync_copy(idx_hbm.at[wid], idx_v)

    def start(c, b):
        pltpu.async_copy(trm_hbm.at[idx_v.at[c]], rows[b], sems[b])

    def finish(c, b):
        pltpu.make_async_copy(trm_hbm.at[idx_v.at[c]], rows[b], sems[b]).wait()
        pltpu.sync_copy(rows[b], out_hbm.at[pl.ds(base + c * _CHUNK, _CHUNK)])

    for b in range(_NBUF):
        start(b, b)

    def step(i, carry):
        c0 = i * _NBUF
        for b in range(_NBUF):
            finish(c0 + b, b)
            start(c0 + b + _NBUF, b)
        return carry

    lax.fori_loop(0, (_N_CHUNKS - _NBUF) // _NBUF, step, 0)

    for b in range(_NBUF):
        finish(_N_CHUNKS - _NBUF + b, b)


def kernel(x, table):
    idx = x.reshape(_NW, _N_CHUNKS, _CHUNK)
    table_t = jnp.swapaxes(table, 0, 1)

    transpose = pl.kernel(
        _transpose_body,
        out_type=jax.ShapeDtypeStruct((_VOCAB * _DIM,), jnp.float32),
        mesh=plsc.VectorSubcoreMesh(core_axis_name="c", subcore_axis_name="s"),
        scratch_types=[
            pltpu.VMEM((_DIM, _W), jnp.float32),
            pltpu.VMEM((_W * _DIM,), jnp.float32),
            pltpu.SemaphoreType.DMA,
        ],
        compiler_params=pltpu.CompilerParams(
            use_tc_tiling_on_sc=False, needs_layout_passes=False
        ),
    )
    table_rm = transpose(table_t).reshape(_VOCAB, _DIM)

    gather = pl.kernel(
        _gather_body,
        out_type=jax.ShapeDtypeStruct((_TOTAL, _DIM), jnp.float32),
        mesh=plsc.VectorSubcoreMesh(core_axis_name="c", subcore_axis_name="s"),
        scratch_types=[
            pltpu.VMEM((_N_CHUNKS, _CHUNK), jnp.int32),
            pltpu.VMEM((_CHUNK, _DIM), jnp.float32),
            pltpu.VMEM((_CHUNK, _DIM), jnp.float32),
            pltpu.SemaphoreType.DMA,
            pltpu.SemaphoreType.DMA,
        ],
        compiler_params=pltpu.CompilerParams(use_tc_tiling_on_sc=False),
    )
    out = gather(idx, table_rm)
    return out.reshape(_BATCH, _FIELDS, _DIM)


# speedup vs baseline: 4.5245x; 4.5245x over previous
import jax
import jax.numpy as jnp
from jax import lax
from jax.experimental import pallas as pl
from jax.experimental.pallas import tpu as pltpu
from jax.experimental.pallas import tpu_sc as plsc

_BATCH = 16384
_FIELDS = 26
_DIM = 32
_VOCAB = 1000000
_TOTAL = _BATCH * _FIELDS
_NUM_CORES = 2
_NUM_SUBCORES = 16
_NW = _NUM_CORES * _NUM_SUBCORES
_B_PER_W = _TOTAL // _NW
_N_CHUNKS = 8
_CHUNK = _B_PER_W // _N_CHUNKS
_NBUF = 2


def _gather_body(idx_hbm, table_hbm, out_hbm, idx_v, rows0, rows1, s0, s1):
    wid = lax.axis_index("s") * _NUM_CORES + lax.axis_index("c")
    base = wid * _B_PER_W
    rows = (rows0, rows1)
    sems = (s0, s1)

    pltpu.sync_copy(idx_hbm.at[pl.ds(base, _B_PER_W)], idx_v)

    def start(c, b):
        pltpu.async_copy(table_hbm.at[idx_v.at[pl.ds(c * _CHUNK, _CHUNK)]], rows[b], sems[b])

    def finish(c, b):
        pltpu.make_async_copy(table_hbm.at[idx_v.at[pl.ds(c * _CHUNK, _CHUNK)]], rows[b], sems[b]).wait()
        pltpu.sync_copy(
            rows[b], out_hbm.at[pl.ds(base + c * _CHUNK, _CHUNK)]
        )

    for b in range(_NBUF):
        start(b, b)

    def step(i, carry):
        c0 = i * _NBUF
        for b in range(_NBUF):
            finish(c0 + b, b)
            start(c0 + b + _NBUF, b)
        return carry

    lax.fori_loop(0, (_N_CHUNKS - _NBUF) // _NBUF, step, 0)

    for b in range(_NBUF):
        finish(_N_CHUNKS - _NBUF + b, b)


def kernel(x, table):
    idx = x.reshape(_TOTAL)
    gather = pl.kernel(
        _gather_body,
        out_type=jax.ShapeDtypeStruct((_TOTAL, _DIM), jnp.float32),
        mesh=plsc.VectorSubcoreMesh(core_axis_name="c", subcore_axis_name="s"),
        scratch_types=[
            pltpu.VMEM((_B_PER_W,), jnp.int32),
            pltpu.VMEM((_CHUNK, _DIM), jnp.float32),
            pltpu.VMEM((_CHUNK, _DIM), jnp.float32),
            pltpu.SemaphoreType.DMA,
            pltpu.SemaphoreType.DMA,
        ],
        compiler_params=pltpu.CompilerParams(use_tc_tiling_on_sc=False),
    )
    out = gather(idx, table)
    return out.reshape(_BATCH, _FIELDS, _DIM)
